# Initial kernel scaffold; baseline (speedup 1.0000x reference)
import jax
import jax.numpy as jnp
from jax.experimental import pallas as pl


def kernel(inputs, W1, b1, w2, p):
    raise NotImplementedError("write your pallas kernel here")


# speedup vs baseline: 2.6029x; 2.6029x over previous
import functools

import numpy as np
import jax
import jax.numpy as jnp
from jax import lax
from jax.experimental import pallas as pl
from jax.experimental.pallas import tpu as pltpu

_F, _D, _A = 50, 64, 64
_P = (_F * (_F - 1)) // 2
_PPAD = 1280

_i0, _i1 = np.triu_indices(_F, k=1)
_R1 = np.zeros((_F, _PPAD), np.float32)
_R1[_i0, np.arange(_P)] = 1.0
_R2 = np.zeros((_F, _PPAD), np.float32)
_R2[_i1, np.arange(_P)] = 1.0
_MNEG = np.full((1, _PPAD), -1e30, np.float32)
_MNEG[0, :_P] = 0.0

_BB = 16


def _afm_body(x_ref, w1e_ref, b1_ref, w2_ref, r1_ref, r2_ref, mneg_ref, o_ref):
    w1e = w1e_ref[...]
    b1c = b1_ref[...]
    w2c = w2_ref[...]
    r1 = r1_ref[...]
    r2 = r2_ref[...]
    mneg = mneg_ref[...]
    dn = (((0,), (0,)), ((), ()))
    f32 = jnp.float32
    for b in range(_BB):
        xb = x_ref[b]
        a1 = lax.dot_general(xb, r1, dn, preferred_element_type=f32)
        a2 = lax.dot_general(xb, r2, dn, preferred_element_type=f32)
        prod_t = a1 * a2
        hq = lax.dot_general(w1e, prod_t, dn, preferred_element_type=f32)
        h_t = jnp.maximum(hq[:_A, :] + b1c, 0.0)
        q = hq[_A:_A + 1, :]
        logits = lax.dot_general(w2c, h_t, dn, preferred_element_type=f32)
        logits = logits + mneg
        m = jnp.max(logits, axis=1, keepdims=True)
        e = jnp.exp(logits - m)
        s = jnp.sum(e, axis=1, keepdims=True)
        num = jnp.sum(e * q, axis=1, keepdims=True)
        o_ref[b, :, :] = num / s


@jax.jit
def _afm(inputs, W1, b1, w2, p):
    B = inputs.shape[0]
    w1e = jnp.concatenate([W1, p[:, None]], axis=1)
    b1c = b1[:, None]
    w2c = w2[:, None]
    r1 = jnp.asarray(_R1)
    r2 = jnp.asarray(_R2)
    mneg = jnp.asarray(_MNEG)
    grid = (B // _BB,)
    out = pl.pallas_call(
        _afm_body,
        grid=grid,
        in_specs=[
            pl.BlockSpec((_BB, _F, _D), lambda i: (i, 0, 0)),
            pl.BlockSpec((_D, _A + 1), lambda i: (0, 0)),
            pl.BlockSpec((_A, 1), lambda i: (0, 0)),
            pl.BlockSpec((_A, 1), lambda i: (0, 0)),
            pl.BlockSpec((_F, _PPAD), lambda i: (0, 0)),
            pl.BlockSpec((_F, _PPAD), lambda i: (0, 0)),
            pl.BlockSpec((1, _PPAD), lambda i: (0, 0)),
        ],
        out_specs=pl.BlockSpec((_BB, 1, 1), lambda i: (i, 0, 0)),
        out_shape=jax.ShapeDtypeStruct((B, 1, 1), jnp.float32),
        compiler_params=pltpu.CompilerParams(
            dimension_semantics=("parallel",),
        ),
    )(inputs, w1e, b1c, w2c, r1, r2, mneg)
    return out.reshape(B)


def kernel(inputs, W1, b1, w2, p):
    return _afm(inputs, W1, b1, w2, p)


# speedup vs baseline: 5.2591x; 2.0205x over previous
import numpy as np
import jax
import jax.numpy as jnp
from jax import lax
from jax.experimental import pallas as pl
from jax.experimental.pallas import tpu as pltpu

_F, _D, _A = 50, 64, 64
_P = (_F * (_F - 1)) // 2
_PPAD = 1280

_i0, _i1 = np.triu_indices(_F, k=1)
_R12 = np.zeros((_F, 2 * _PPAD), np.float32)
_R12[_i0, np.arange(_P)] = 1.0
_R12[_i1, _PPAD + np.arange(_P)] = 1.0

_BB = 16

_MNEG = np.full((1, _PPAD), -1e30, np.float32)
_MNEG[0, :_P] = 0.0
_MNEG_T = np.tile(_MNEG, (1, _BB))


def _afm_body(x_ref, w1e_ref, b1_ref, w2_ref, r12_ref, mneg_ref, o_ref):
    w1e = w1e_ref[...]
    b1c = b1_ref[...]
    w2c = w2_ref[...]
    r12 = r12_ref[...]
    mneg = mneg_ref[...]
    dn = (((0,), (0,)), ((), ()))
    f32 = jnp.float32
    bf16 = jnp.bfloat16

    prods = []
    for b in range(_BB):
        xb = x_ref[b].astype(bf16)
        a12 = lax.dot_general(xb, r12, dn, preferred_element_type=f32)
        prods.append((a12[:, :_PPAD] * a12[:, _PPAD:]).astype(bf16))
    prod_t = jnp.concatenate(prods, axis=1)

    hq = lax.dot_general(w1e, prod_t, dn, preferred_element_type=f32)
    h_t = jnp.maximum(hq[:_A, :] + b1c, 0.0).astype(bf16)
    q = hq[_A:_A + 1, :]

    logits = lax.dot_general(w2c, h_t, dn, preferred_element_type=f32)
    logits = logits + mneg

    for b in range(_BB):
        lg = logits[:, b * _PPAD:(b + 1) * _PPAD]
        qb = q[:, b * _PPAD:(b + 1) * _PPAD]
        m = jnp.max(lg, axis=1, keepdims=True)
        e = jnp.exp(lg - m)
        s = jnp.sum(e, axis=1, keepdims=True)
        num = jnp.sum(e * qb, axis=1, keepdims=True)
        o_ref[b, :, :] = num / s


@jax.jit
def _afm(inputs, W1, b1, w2, p):
    B = inputs.shape[0]
    w1e = jnp.concatenate([W1, p[:, None]], axis=1).astype(jnp.bfloat16)
    b1c = b1[:, None]
    w2c = w2[:, None].astype(jnp.bfloat16)
    r12 = jnp.asarray(_R12).astype(jnp.bfloat16)
    mneg = jnp.asarray(_MNEG_T)
    grid = (B // _BB,)
    out = pl.pallas_call(
        _afm_body,
        grid=grid,
        in_specs=[
            pl.BlockSpec((_BB, _F, _D), lambda i: (i, 0, 0)),
            pl.BlockSpec((_D, _A + 1), lambda i: (0, 0)),
            pl.BlockSpec((_A, 1), lambda i: (0, 0)),
            pl.BlockSpec((_A, 1), lambda i: (0, 0)),
            pl.BlockSpec((_F, 2 * _PPAD), lambda i: (0, 0)),
            pl.BlockSpec((1, _BB * _PPAD), lambda i: (0, 0)),
        ],
        out_specs=pl.BlockSpec((_BB, 1, 1), lambda i: (i, 0, 0)),
        out_shape=jax.ShapeDtypeStruct((B, 1, 1), jnp.float32),
        compiler_params=pltpu.CompilerParams(
            dimension_semantics=(pltpu.PARALLEL,),
        ),
    )(inputs, w1e, b1c, w2c, r12, mneg)
    return out.reshape(B)


def kernel(inputs, W1, b1, w2, p):
    return _afm(inputs, W1, b1, w2, p)


# speedup vs baseline: 5.5871x; 1.0624x over previous
import numpy as np
import jax
import jax.numpy as jnp
from jax import lax
from jax.experimental import pallas as pl
from jax.experimental.pallas import tpu as pltpu

_F, _D, _A = 50, 64, 64
_P = (_F * (_F - 1)) // 2
_PPAD = 1280

_i0, _i1 = np.triu_indices(_F, k=1)
_R12 = np.zeros((_F, 2 * _PPAD), np.float32)
_R12[_i0, np.arange(_P)] = 1.0
_R12[_i1, _PPAD + np.arange(_P)] = 1.0

_BB = 32

_MNEG = np.full((1, _PPAD), -1e30, np.float32)
_MNEG[0, :_P] = 0.0
_MNEG_T = np.tile(_MNEG, (1, _BB))


def _afm_body(x_ref, w1e_ref, b1_ref, w2_ref, r12_ref, mneg_ref, o_ref):
    w1e = w1e_ref[...]
    b1c = b1_ref[...]
    w2c = w2_ref[...]
    r12 = r12_ref[...]
    mneg = mneg_ref[...]
    dn = (((0,), (0,)), ((), ()))
    f32 = jnp.float32
    bf16 = jnp.bfloat16

    prods = []
    for b in range(_BB):
        xb = x_ref[b].astype(bf16)
        a12 = lax.dot_general(xb, r12, dn, preferred_element_type=f32)
        prods.append((a12[:, :_PPAD] * a12[:, _PPAD:]).astype(bf16))
    prod_t = jnp.concatenate(prods, axis=1)

    hq = lax.dot_general(w1e, prod_t, dn, preferred_element_type=f32)
    h_t = jnp.maximum(hq[:_A, :] + b1c, 0.0).astype(bf16)
    q = hq[_A:_A + 1, :]

    logits = lax.dot_general(w2c, h_t, dn, preferred_element_type=f32)
    logits = logits + mneg

    for b in range(_BB):
        lg = logits[:, b * _PPAD:(b + 1) * _PPAD]
        qb = q[:, b * _PPAD:(b + 1) * _PPAD]
        m = jnp.max(lg, axis=1, keepdims=True)
        e = jnp.exp(lg - m)
        s = jnp.sum(e, axis=1, keepdims=True)
        num = jnp.sum(e * qb, axis=1, keepdims=True)
        o_ref[b, :, :] = num / s


@jax.jit
def _afm(inputs, W1, b1, w2, p):
    B = inputs.shape[0]
    w1e = jnp.concatenate([W1, p[:, None]], axis=1).astype(jnp.bfloat16)
    b1c = b1[:, None]
    w2c = w2[:, None].astype(jnp.bfloat16)
    r12 = jnp.asarray(_R12).astype(jnp.bfloat16)
    mneg = jnp.asarray(_MNEG_T)
    grid = (B // _BB,)
    out = pl.pallas_call(
        _afm_body,
        grid=grid,
        in_specs=[
            pl.BlockSpec((_BB, _F, _D), lambda i: (i, 0, 0)),
            pl.BlockSpec((_D, _A + 1), lambda i: (0, 0)),
            pl.BlockSpec((_A, 1), lambda i: (0, 0)),
            pl.BlockSpec((_A, 1), lambda i: (0, 0)),
            pl.BlockSpec((_F, 2 * _PPAD), lambda i: (0, 0)),
            pl.BlockSpec((1, _BB * _PPAD), lambda i: (0, 0)),
        ],
        out_specs=pl.BlockSpec((_BB, 1, 1), lambda i: (i, 0, 0)),
        out_shape=jax.ShapeDtypeStruct((B, 1, 1), jnp.float32),
        compiler_params=pltpu.CompilerParams(
            dimension_semantics=(pltpu.PARALLEL,),
            vmem_limit_bytes=56 * 1024 * 1024,
        ),
    )(inputs, w1e, b1c, w2c, r12, mneg)
    return out.reshape(B)


def kernel(inputs, W1, b1, w2, p):
    return _afm(inputs, W1, b1, w2, p)


# speedup vs baseline: 5.7583x; 1.0306x over previous
import numpy as np
import jax
import jax.numpy as jnp
from jax import lax
from jax.experimental import pallas as pl
from jax.experimental.pallas import tpu as pltpu

_F, _D, _A = 50, 64, 64
_P = (_F * (_F - 1)) // 2
_PPAD = 1280

_i0, _i1 = np.triu_indices(_F, k=1)
_R12 = np.zeros((_F, 2 * _PPAD), np.float32)
_R12[_i0, np.arange(_P)] = 1.0
_R12[_i1, _PPAD + np.arange(_P)] = 1.0

_BB = 64

_MNEG = np.full((1, _PPAD), -1e30, np.float32)
_MNEG[0, :_P] = 0.0
_MNEG_T = np.tile(_MNEG, (1, _BB))


def _afm_body(x_ref, w1e_ref, b1_ref, w2_ref, r12_ref, mneg_ref, o_ref):
    w1e = w1e_ref[...]
    b1c = b1_ref[...]
    w2c = w2_ref[...]
    r12 = r12_ref[...]
    mneg = mneg_ref[...]
    dn = (((0,), (0,)), ((), ()))
    f32 = jnp.float32
    bf16 = jnp.bfloat16

    prods = []
    for b in range(_BB):
        xb = x_ref[b].astype(bf16)
        a12 = lax.dot_general(xb, r12, dn, preferred_element_type=f32)
        prods.append((a12[:, :_PPAD] * a12[:, _PPAD:]).astype(bf16))
    prod_t = jnp.concatenate(prods, axis=1)

    hq = lax.dot_general(w1e, prod_t, dn, preferred_element_type=f32)
    h_t = jnp.maximum(hq[:_A, :] + b1c, 0.0).astype(bf16)
    q = hq[_A:_A + 1, :]

    logits = lax.dot_general(w2c, h_t, dn, preferred_element_type=f32)
    logits = logits + mneg

    for b in range(_BB):
        lg = logits[:, b * _PPAD:(b + 1) * _PPAD]
        qb = q[:, b * _PPAD:(b + 1) * _PPAD]
        m = jnp.max(lg, axis=1, keepdims=True)
        e = jnp.exp(lg - m)
        s = jnp.sum(e, axis=1, keepdims=True)
        num = jnp.sum(e * qb, axis=1, keepdims=True)
        o_ref[b, :, :] = num / s


@jax.jit
def _afm(inputs, W1, b1, w2, p):
    B = inputs.shape[0]
    w1e = jnp.concatenate([W1, p[:, None]], axis=1).astype(jnp.bfloat16)
    b1c = b1[:, None]
    w2c = w2[:, None].astype(jnp.bfloat16)
    r12 = jnp.asarray(_R12).astype(jnp.bfloat16)
    mneg = jnp.asarray(_MNEG_T)
    grid = (B // _BB,)
    out = pl.pallas_call(
        _afm_body,
        grid=grid,
        in_specs=[
            pl.BlockSpec((_BB, _F, _D), lambda i: (i, 0, 0)),
            pl.BlockSpec((_D, _A + 1), lambda i: (0, 0)),
            pl.BlockSpec((_A, 1), lambda i: (0, 0)),
            pl.BlockSpec((_A, 1), lambda i: (0, 0)),
            pl.BlockSpec((_F, 2 * _PPAD), lambda i: (0, 0)),
            pl.BlockSpec((1, _BB * _PPAD), lambda i: (0, 0)),
        ],
        out_specs=pl.BlockSpec((_BB, 1, 1), lambda i: (i, 0, 0)),
        out_shape=jax.ShapeDtypeStruct((B, 1, 1), jnp.float32),
        compiler_params=pltpu.CompilerParams(
            dimension_semantics=(pltpu.PARALLEL,),
            vmem_limit_bytes=56 * 1024 * 1024,
        ),
    )(inputs, w1e, b1c, w2c, r12, mneg)
    return out.reshape(B)


def kernel(inputs, W1, b1, w2, p):
    return _afm(inputs, W1, b1, w2, p)
